# baseline (device time: 45885 ns/iter reference)
import jax
import jax.numpy as jnp
from jax import lax
from jax.experimental import pallas as pl
from jax.experimental.pallas import tpu as pltpu

N_DEV = 32
LOG2 = 5
N_TOK = 256
D_IN = 128
D_OUT = 256
N_EXP = 64
E_LOC = 2


def kernel(x, router_W, route_idx, expert_W):
    def body(x_ref, rw_ref, idx_ref, ew_ref, out_ref, recv_buf, send_sems, recv_sems):
        my = lax.axis_index("i")

        xs = x_ref[:, :]
        scores = jnp.dot(xs, rw_ref[:, :], preferred_element_type=jnp.float32)
        m = jnp.max(scores, axis=-1, keepdims=True)
        p = jnp.exp(scores - m)
        p = p / jnp.sum(p, axis=-1, keepdims=True)

        idx0 = idx_ref[:, 0:1]
        idx1 = idx_ref[:, 1:2]
        eids = lax.broadcasted_iota(jnp.int32, (N_TOK, N_EXP), 1)
        g0 = jnp.sum(jnp.where(idx0 == eids, p, 0.0), axis=1, keepdims=True)
        g1 = jnp.sum(jnp.where(idx1 == eids, p, 0.0), axis=1, keepdims=True)
        gs = g0 + g1

        xb = xs.astype(jnp.bfloat16)
        acc = jnp.zeros((N_TOK, D_OUT), jnp.float32)
        for j in range(E_LOC):
            gid = my * E_LOC + j
            wj = jnp.where(idx0 == gid, g0 / gs, 0.0) + jnp.where(
                idx1 == gid, g1 / gs, 0.0
            )
            yj = jnp.dot(
                xb,
                ew_ref[j, :, :].astype(jnp.bfloat16),
                preferred_element_type=jnp.float32,
            )
            acc = acc + wj * yj
        out_ref[:, :] = acc

        for k in range(LOG2):
            peer = my ^ (1 << k)
            rdma = pltpu.make_async_remote_copy(
                src_ref=out_ref,
                dst_ref=recv_buf.at[k],
                send_sem=send_sems.at[k],
                recv_sem=recv_sems.at[k],
                device_id=(peer,),
                device_id_type=pl.DeviceIdType.MESH,
            )
            rdma.start()
            rdma.wait()
            out_ref[:, :] = out_ref[:, :] + recv_buf[k, :, :]

    return pl.pallas_call(
        body,
        out_shape=jax.ShapeDtypeStruct((N_TOK, D_OUT), jnp.float32),
        in_specs=[pl.BlockSpec(memory_space=pltpu.VMEM)] * 4,
        out_specs=pl.BlockSpec(memory_space=pltpu.VMEM),
        scratch_shapes=[
            pltpu.VMEM((LOG2, N_TOK, D_OUT), jnp.float32),
            pltpu.SemaphoreType.DMA((LOG2,)),
            pltpu.SemaphoreType.DMA((LOG2,)),
        ],
    )(x, router_W, route_idx, expert_W)


# device time: 25816 ns/iter; 1.7774x vs baseline; 1.7774x over previous
import jax
import jax.numpy as jnp
from jax import lax
from jax.experimental import pallas as pl
from jax.experimental.pallas import tpu as pltpu

N_DEV = 32
N_TOK = 256
D_IN = 128
D_OUT = 256
N_EXP = 64
E_LOC = 2
R_SL = N_TOK // N_DEV


def kernel(x, router_W, route_idx, expert_W):
    def body(
        x_ref,
        rw_ref,
        idx_ref,
        ew_ref,
        out_ref,
        pbuf,
        rbuf,
        recv1,
        recv2,
        send_sems1,
        recv_sems1,
        send_sems2,
        recv_sems2,
    ):
        my = lax.axis_index("i")

        xs = x_ref[:, :]
        scores = jnp.dot(xs, rw_ref[:, :], preferred_element_type=jnp.float32)
        m = jnp.max(scores, axis=-1, keepdims=True)
        p = jnp.exp(scores - m)
        p = p / jnp.sum(p, axis=-1, keepdims=True)

        idx0 = idx_ref[:, 0:1]
        idx1 = idx_ref[:, 1:2]
        eids = lax.broadcasted_iota(jnp.int32, (N_TOK, N_EXP), 1)
        g0 = jnp.sum(jnp.where(idx0 == eids, p, 0.0), axis=1, keepdims=True)
        g1 = jnp.sum(jnp.where(idx1 == eids, p, 0.0), axis=1, keepdims=True)
        gs = g0 + g1

        xb = xs.astype(jnp.bfloat16)
        acc = jnp.zeros((N_TOK, D_OUT), jnp.float32)
        for j in range(E_LOC):
            gid = my * E_LOC + j
            wj = jnp.where(idx0 == gid, g0 / gs, 0.0) + jnp.where(
                idx1 == gid, g1 / gs, 0.0
            )
            yj = jnp.dot(
                xb,
                ew_ref[j, :, :].astype(jnp.bfloat16),
                preferred_element_type=jnp.float32,
            )
            acc = acc + wj * yj
        pbuf[:, :] = acc.astype(jnp.bfloat16)

        sends1 = []
        for o in range(1, N_DEV):
            peer = (my + o) & (N_DEV - 1)
            rdma = pltpu.make_async_remote_copy(
                src_ref=pbuf.at[pl.ds(peer * R_SL, R_SL), :],
                dst_ref=recv1.at[my],
                send_sem=send_sems1.at[o - 1],
                recv_sem=recv_sems1.at[my],
                device_id=(peer,),
                device_id_type=pl.DeviceIdType.MESH,
            )
            rdma.start()
            sends1.append(rdma)

        recv1[my] = pbuf[pl.ds(my * R_SL, R_SL), :]

        for o in range(1, N_DEV):
            sender = (my + o) & (N_DEV - 1)
            pltpu.make_async_remote_copy(
                src_ref=pbuf.at[pl.ds(0, R_SL), :],
                dst_ref=recv1.at[sender],
                send_sem=send_sems1.at[o - 1],
                recv_sem=recv_sems1.at[sender],
                device_id=(sender,),
                device_id_type=pl.DeviceIdType.MESH,
            ).wait_recv()

        rbuf[:, :] = jnp.sum(
            recv1[...].astype(jnp.float32), axis=0
        ).astype(jnp.bfloat16)

        sends2 = []
        for o in range(1, N_DEV):
            peer = (my + o) & (N_DEV - 1)
            rdma = pltpu.make_async_remote_copy(
                src_ref=rbuf,
                dst_ref=recv2.at[my],
                send_sem=send_sems2.at[o - 1],
                recv_sem=recv_sems2.at[my],
                device_id=(peer,),
                device_id_type=pl.DeviceIdType.MESH,
            )
            rdma.start()
            sends2.append(rdma)

        recv2[my] = rbuf[:, :]

        for o in range(1, N_DEV):
            sender = (my + o) & (N_DEV - 1)
            pltpu.make_async_remote_copy(
                src_ref=rbuf,
                dst_ref=recv2.at[sender],
                send_sem=send_sems2.at[o - 1],
                recv_sem=recv_sems2.at[sender],
                device_id=(sender,),
                device_id_type=pl.DeviceIdType.MESH,
            ).wait_recv()

        out_ref[:, :] = (
            recv2[...].astype(jnp.float32).reshape(N_TOK, D_OUT)
        )

        for rdma in sends1:
            rdma.wait_send()
        for rdma in sends2:
            rdma.wait_send()

    return pl.pallas_call(
        body,
        out_shape=jax.ShapeDtypeStruct((N_TOK, D_OUT), jnp.float32),
        in_specs=[pl.BlockSpec(memory_space=pltpu.VMEM)] * 4,
        out_specs=pl.BlockSpec(memory_space=pltpu.VMEM),
        scratch_shapes=[
            pltpu.VMEM((N_TOK, D_OUT), jnp.bfloat16),
            pltpu.VMEM((R_SL, D_OUT), jnp.bfloat16),
            pltpu.VMEM((N_DEV, R_SL, D_OUT), jnp.bfloat16),
            pltpu.VMEM((N_DEV, R_SL, D_OUT), jnp.bfloat16),
            pltpu.SemaphoreType.DMA((N_DEV - 1,)),
            pltpu.SemaphoreType.DMA((N_DEV,)),
            pltpu.SemaphoreType.DMA((N_DEV - 1,)),
            pltpu.SemaphoreType.DMA((N_DEV,)),
        ],
    )(x, router_W, route_idx, expert_W)


# device time: 17521 ns/iter; 2.6189x vs baseline; 1.4734x over previous
import jax
import jax.numpy as jnp
from jax import lax
from jax.experimental import pallas as pl
from jax.experimental.pallas import tpu as pltpu

N_DEV = 32
N_TOK = 256
D_IN = 128
D_OUT = 256
N_EXP = 64
E_LOC = 2
R_SL = N_TOK // N_DEV


def kernel(x, router_W, route_idx, expert_W):
    def body(
        x_ref,
        rw_ref,
        idx_ref,
        ew_ref,
        out_ref,
        pbuf,
        rbuf,
        recv1,
        recv2,
        send_sems1,
        recv_sems1,
        send_sems2,
        recv_sems2,
    ):
        my = lax.axis_index("i")

        barrier_sem = pltpu.get_barrier_semaphore()
        for o in range(1, N_DEV):
            peer = (my + o) & (N_DEV - 1)
            pl.semaphore_signal(
                barrier_sem,
                inc=1,
                device_id=(peer,),
                device_id_type=pl.DeviceIdType.MESH,
            )

        xs = x_ref[:, :]
        scores = jnp.dot(xs, rw_ref[:, :], preferred_element_type=jnp.float32)
        m = jnp.max(scores, axis=-1, keepdims=True)
        p = jnp.exp(scores - m)
        p = p / jnp.sum(p, axis=-1, keepdims=True)

        idx0 = idx_ref[:, 0:1]
        idx1 = idx_ref[:, 1:2]
        eids = lax.broadcasted_iota(jnp.int32, (N_TOK, N_EXP), 1)
        g0 = jnp.sum(jnp.where(idx0 == eids, p, 0.0), axis=1, keepdims=True)
        g1 = jnp.sum(jnp.where(idx1 == eids, p, 0.0), axis=1, keepdims=True)
        gs = g0 + g1

        xb = xs.astype(jnp.bfloat16)
        acc = jnp.zeros((N_TOK, D_OUT), jnp.float32)
        for j in range(E_LOC):
            gid = my * E_LOC + j
            wj = jnp.where(idx0 == gid, g0 / gs, 0.0) + jnp.where(
                idx1 == gid, g1 / gs, 0.0
            )
            yj = jnp.dot(
                xb,
                ew_ref[j, :, :].astype(jnp.bfloat16),
                preferred_element_type=jnp.float32,
            )
            acc = acc + wj * yj
        pbuf[:, :] = acc.astype(jnp.bfloat16)

        pl.semaphore_wait(barrier_sem, N_DEV - 1)

        sends1 = []
        for o in range(1, N_DEV):
            peer = (my + o) & (N_DEV - 1)
            rdma = pltpu.make_async_remote_copy(
                src_ref=pbuf.at[pl.ds(peer * R_SL, R_SL), :],
                dst_ref=recv1.at[my],
                send_sem=send_sems1.at[o - 1],
                recv_sem=recv_sems1.at[my],
                device_id=(peer,),
                device_id_type=pl.DeviceIdType.MESH,
            )
            rdma.start()
            sends1.append(rdma)

        recv1[my] = pbuf[pl.ds(my * R_SL, R_SL), :]

        for o in range(1, N_DEV):
            sender = (my + o) & (N_DEV - 1)
            pltpu.make_async_remote_copy(
                src_ref=pbuf.at[pl.ds(0, R_SL), :],
                dst_ref=recv1.at[sender],
                send_sem=send_sems1.at[o - 1],
                recv_sem=recv_sems1.at[sender],
                device_id=(sender,),
                device_id_type=pl.DeviceIdType.MESH,
            ).wait_recv()

        rbuf[:, :] = jnp.sum(
            recv1[...].astype(jnp.float32), axis=0
        ).astype(jnp.bfloat16)

        sends2 = []
        for o in range(1, N_DEV):
            peer = (my + o) & (N_DEV - 1)
            rdma = pltpu.make_async_remote_copy(
                src_ref=rbuf,
                dst_ref=recv2.at[my],
                send_sem=send_sems2.at[o - 1],
                recv_sem=recv_sems2.at[my],
                device_id=(peer,),
                device_id_type=pl.DeviceIdType.MESH,
            )
            rdma.start()
            sends2.append(rdma)

        recv2[my] = rbuf[:, :]

        for o in range(1, N_DEV):
            sender = (my + o) & (N_DEV - 1)
            pltpu.make_async_remote_copy(
                src_ref=rbuf,
                dst_ref=recv2.at[sender],
                send_sem=send_sems2.at[o - 1],
                recv_sem=recv_sems2.at[sender],
                device_id=(sender,),
                device_id_type=pl.DeviceIdType.MESH,
            ).wait_recv()

        out_ref[:, :] = (
            recv2[...].astype(jnp.float32).reshape(N_TOK, D_OUT)
        )

        for rdma in sends1:
            rdma.wait_send()
        for rdma in sends2:
            rdma.wait_send()

    return pl.pallas_call(
        body,
        out_shape=jax.ShapeDtypeStruct((N_TOK, D_OUT), jnp.float32),
        in_specs=[pl.BlockSpec(memory_space=pltpu.VMEM)] * 4,
        out_specs=pl.BlockSpec(memory_space=pltpu.VMEM),
        compiler_params=pltpu.CompilerParams(collective_id=0),
        scratch_shapes=[
            pltpu.VMEM((N_TOK, D_OUT), jnp.bfloat16),
            pltpu.VMEM((R_SL, D_OUT), jnp.bfloat16),
            pltpu.VMEM((N_DEV, R_SL, D_OUT), jnp.bfloat16),
            pltpu.VMEM((N_DEV, R_SL, D_OUT), jnp.bfloat16),
            pltpu.SemaphoreType.DMA((N_DEV - 1,)),
            pltpu.SemaphoreType.DMA((N_DEV,)),
            pltpu.SemaphoreType.DMA((N_DEV - 1,)),
            pltpu.SemaphoreType.DMA((N_DEV,)),
        ],
    )(x, router_W, route_idx, expert_W)
